# baseline (speedup 1.0000x reference)
import functools

import jax
import jax.numpy as jnp
from jax import lax
from jax.experimental import pallas as pl
from jax.experimental.pallas import tpu as pltpu
from jax.experimental.pallas import tpu_sc as plsc

_NUM_E = 1000000
_DIM = 64
_WPG = _DIM // 4
_GROUP = 128
_SUPER = 8
_NW = 32
_ROWS_TC = 512


def _sc_gather(w32, idx2d, scales):
    groups_total = idx2d.shape[0]
    groups_per_w = groups_total // _NW
    supers_per_w = groups_per_w // _SUPER
    rows_per_super = _SUPER * _GROUP
    b_flat = groups_total * _GROUP

    mesh = plsc.VectorSubcoreMesh(core_axis_name="c", subcore_axis_name="s")

    @functools.partial(
        pl.kernel,
        mesh=mesh,
        out_type=[
            jax.ShapeDtypeStruct((b_flat, _WPG), jnp.int32),
            jax.ShapeDtypeStruct((b_flat,), jnp.float32),
        ],
        scratch_types=[
            pltpu.VMEM((groups_per_w, _GROUP), jnp.int32),
            pltpu.VMEM((rows_per_super, _WPG), jnp.int32),
            pltpu.VMEM((rows_per_super,), jnp.float32),
            pltpu.SemaphoreType.DMA,
            pltpu.SemaphoreType.DMA,
        ],
        compiler_params=pltpu.CompilerParams(use_tc_tiling_on_sc=False),
    )
    def k(w_hbm, idx_hbm, s_hbm, words_hbm, svals_hbm, idx_v, rows_v, sc_v,
          sem_w, sem_s):
        wid = lax.axis_index("s") * 2 + lax.axis_index("c")
        g0 = wid * groups_per_w
        pltpu.sync_copy(idx_hbm.at[pl.ds(g0, groups_per_w)], idx_v)

        def super_body(sg, carry):
            copies = []
            for j in range(_SUPER):
                row_idx = idx_v.at[sg * _SUPER + j]
                copies.append(pltpu.async_copy(
                    w_hbm.at[row_idx], rows_v.at[pl.ds(j * _GROUP, _GROUP)],
                    sem_w))
                copies.append(pltpu.async_copy(
                    s_hbm.at[row_idx], sc_v.at[pl.ds(j * _GROUP, _GROUP)],
                    sem_s))
            for cp in copies:
                cp.wait()
            base = (g0 + sg * _SUPER) * _GROUP
            pltpu.sync_copy(rows_v, words_hbm.at[pl.ds(base, rows_per_super)])
            pltpu.sync_copy(sc_v, svals_hbm.at[pl.ds(base, rows_per_super)])
            return carry

        lax.fori_loop(0, supers_per_w, super_body, 0)

    return k(w32, idx2d, scales)


def _tc_dequant_body(x_ref, s_ref, o_ref):
    xa = x_ref[...].astype(jnp.float32)
    s = s_ref[...]
    o_ref[:, :_DIM] = xa[:, :_DIM] * s[:, 0:1]
    o_ref[:, _DIM:] = xa[:, _DIM:] * s[:, 1:2]


def _tc_dequant(x8, s2):
    n = x8.shape[0]
    grid = (n // _ROWS_TC,)
    return pl.pallas_call(
        _tc_dequant_body,
        grid=grid,
        in_specs=[
            pl.BlockSpec((_ROWS_TC, 2 * _DIM), lambda i: (i, 0)),
            pl.BlockSpec((_ROWS_TC, 2), lambda i: (i, 0)),
        ],
        out_specs=pl.BlockSpec((_ROWS_TC, 2 * _DIM), lambda i: (i, 0)),
        out_shape=jax.ShapeDtypeStruct((n, 2 * _DIM), jnp.float32),
        compiler_params=pltpu.CompilerParams(
            dimension_semantics=("arbitrary",)),
    )(x8, s2)


def kernel(x, weights, scales):
    batch, hist = x.shape
    b_flat = batch * hist
    groups_total = b_flat // _GROUP
    idx2d = x.reshape(groups_total, _GROUP)
    w32 = lax.bitcast_convert_type(
        weights.reshape(_NUM_E, _WPG, 4), jnp.int32)
    words, svals = _sc_gather(w32, idx2d, scales)
    x8 = lax.bitcast_convert_type(words, jnp.int8).reshape(b_flat // 2,
                                                           2 * _DIM)
    s2 = svals.reshape(b_flat // 2, 2)
    out = _tc_dequant(x8, s2)
    return out.reshape(batch, hist, _DIM)


# speedup vs baseline: 5.1357x; 5.1357x over previous
import functools

import jax
import jax.numpy as jnp
from jax import lax
from jax.experimental import pallas as pl
from jax.experimental.pallas import tpu as pltpu
from jax.experimental.pallas import tpu_sc as plsc

_NUM_E = 1000000
_DIM = 64
_WPG = _DIM // 4
_GROUP = 128
_SUPER = 8
_NW = 32
_ROWS_TC = 512


def _sc_gather(w8, idx2d, scales):
    groups_total = idx2d.shape[0]
    groups_per_w = groups_total // _NW
    supers_per_w = groups_per_w // _SUPER
    rows_per_super = _SUPER * _GROUP
    b_flat = groups_total * _GROUP

    mesh = plsc.VectorSubcoreMesh(core_axis_name="c", subcore_axis_name="s")

    @functools.partial(
        pl.kernel,
        mesh=mesh,
        out_type=[
            jax.ShapeDtypeStruct((b_flat, _DIM), jnp.int8),
            jax.ShapeDtypeStruct((b_flat,), jnp.float32),
        ],
        scratch_types=[
            pltpu.VMEM((groups_per_w, _GROUP), jnp.int32),
            pltpu.VMEM((rows_per_super, _DIM), jnp.int8),
            pltpu.VMEM((rows_per_super,), jnp.float32),
            pltpu.SemaphoreType.DMA,
            pltpu.SemaphoreType.DMA,
        ],
        compiler_params=pltpu.CompilerParams(use_tc_tiling_on_sc=False),
    )
    def k(w_hbm, idx_hbm, s_hbm, words_hbm, svals_hbm, idx_v, rows_v, sc_v,
          sem_w, sem_s):
        wid = lax.axis_index("s") * 2 + lax.axis_index("c")
        g0 = wid * groups_per_w
        pltpu.sync_copy(idx_hbm.at[pl.ds(g0, groups_per_w)], idx_v)

        def super_body(sg, carry):
            copies = []
            for j in range(_SUPER):
                row_idx = idx_v.at[sg * _SUPER + j]
                copies.append(pltpu.async_copy(
                    w_hbm.at[row_idx], rows_v.at[pl.ds(j * _GROUP, _GROUP)],
                    sem_w))
                copies.append(pltpu.async_copy(
                    s_hbm.at[row_idx], sc_v.at[pl.ds(j * _GROUP, _GROUP)],
                    sem_s))
            for cp in copies:
                cp.wait()
            base = (g0 + sg * _SUPER) * _GROUP
            pltpu.sync_copy(rows_v, words_hbm.at[pl.ds(base, rows_per_super)])
            pltpu.sync_copy(sc_v, svals_hbm.at[pl.ds(base, rows_per_super)])
            return carry

        lax.fori_loop(0, supers_per_w, super_body, 0)

    return k(w8, idx2d, scales)


def _tc_dequant_body(x_ref, s_ref, o_ref):
    xa = x_ref[...].astype(jnp.float32)
    s = s_ref[...]
    o_ref[:, :_DIM] = xa[:, :_DIM] * s[:, 0:1]
    o_ref[:, _DIM:] = xa[:, _DIM:] * s[:, 1:2]


def _tc_dequant(x8, s2):
    n = x8.shape[0]
    grid = (n // _ROWS_TC,)
    return pl.pallas_call(
        _tc_dequant_body,
        grid=grid,
        in_specs=[
            pl.BlockSpec((_ROWS_TC, 2 * _DIM), lambda i: (i, 0)),
            pl.BlockSpec((_ROWS_TC, 2), lambda i: (i, 0)),
        ],
        out_specs=pl.BlockSpec((_ROWS_TC, 2 * _DIM), lambda i: (i, 0)),
        out_shape=jax.ShapeDtypeStruct((n, 2 * _DIM), jnp.float32),
        compiler_params=pltpu.CompilerParams(
            dimension_semantics=("arbitrary",)),
    )(x8, s2)


def kernel(x, weights, scales):
    batch, hist = x.shape
    b_flat = batch * hist
    groups_total = b_flat // _GROUP
    idx2d = x.reshape(groups_total, _GROUP)
    rows8, svals = _sc_gather(weights, idx2d, scales)
    x8 = rows8.reshape(b_flat // 2, 2 * _DIM)
    s2 = svals.reshape(b_flat // 2, 2)
    out = _tc_dequant(x8, s2)
    return out.reshape(batch, hist, _DIM)


# speedup vs baseline: 5.1667x; 1.0060x over previous
import functools

import jax
import jax.numpy as jnp
from jax import lax
from jax.experimental import pallas as pl
from jax.experimental.pallas import tpu as pltpu
from jax.experimental.pallas import tpu_sc as plsc

_NUM_E = 1000000
_DIM = 64
_WPG = _DIM // 4
_GROUP = 128
_SUPER = 8
_NW = 32
_ROWS_TC = 512


def _sc_gather(w8, idx2d, scales):
    groups_total = idx2d.shape[0]
    groups_per_w = groups_total // _NW
    supers_per_w = groups_per_w // _SUPER
    rows_per_super = _SUPER * _GROUP
    b_flat = groups_total * _GROUP

    mesh = plsc.VectorSubcoreMesh(core_axis_name="c", subcore_axis_name="s")

    @functools.partial(
        pl.kernel,
        mesh=mesh,
        out_type=[
            jax.ShapeDtypeStruct((b_flat, _DIM), jnp.int8),
            jax.ShapeDtypeStruct((b_flat,), jnp.float32),
        ],
        scratch_types=[
            pltpu.VMEM((groups_per_w, _GROUP), jnp.int32),
            pltpu.VMEM((rows_per_super, _DIM), jnp.int8),
            pltpu.VMEM((rows_per_super,), jnp.float32),
            pltpu.SemaphoreType.DMA,
            pltpu.SemaphoreType.DMA,
        ],
        compiler_params=pltpu.CompilerParams(use_tc_tiling_on_sc=False),
    )
    def k(w_hbm, idx_hbm, s_hbm, words_hbm, svals_hbm, idx_v, rows_v, sc_v,
          sem_w, sem_s):
        wid = lax.axis_index("s") * 2 + lax.axis_index("c")
        g0 = wid * groups_per_w
        pltpu.sync_copy(idx_hbm.at[pl.ds(g0, groups_per_w)], idx_v)

        def super_body(sg, carry):
            copies = []
            for j in range(_SUPER):
                row_idx = idx_v.at[sg * _SUPER + j]
                copies.append(pltpu.async_copy(
                    w_hbm.at[row_idx], rows_v.at[pl.ds(j * _GROUP, _GROUP)],
                    sem_w))
                copies.append(pltpu.async_copy(
                    s_hbm.at[row_idx], sc_v.at[pl.ds(j * _GROUP, _GROUP)],
                    sem_s))
            for cp in copies:
                cp.wait()
            base = (g0 + sg * _SUPER) * _GROUP
            pltpu.sync_copy(rows_v, words_hbm.at[pl.ds(base, rows_per_super)])
            pltpu.sync_copy(sc_v, svals_hbm.at[pl.ds(base, rows_per_super)])
            return carry

        lax.fori_loop(0, supers_per_w, super_body, 0)

    return k(w8, idx2d, scales)


def _tc_dequant_body(x_ref, s_ref, o_ref):
    xa = x_ref[...].astype(jnp.float32)
    s = s_ref[...]
    r2 = lax.broadcasted_iota(jnp.int32, (2, 2 * _DIM), 0)
    c2 = lax.broadcasted_iota(jnp.int32, (2, 2 * _DIM), 1)
    expand = (c2 // _DIM == r2).astype(jnp.float32)
    s128 = jnp.dot(s, expand, preferred_element_type=jnp.float32)
    o_ref[...] = xa * s128


def _tc_dequant(x8, s2):
    n = x8.shape[0]
    grid = (n // _ROWS_TC,)
    return pl.pallas_call(
        _tc_dequant_body,
        grid=grid,
        in_specs=[
            pl.BlockSpec((_ROWS_TC, 2 * _DIM), lambda i: (i, 0)),
            pl.BlockSpec((_ROWS_TC, 2), lambda i: (i, 0)),
        ],
        out_specs=pl.BlockSpec((_ROWS_TC, 2 * _DIM), lambda i: (i, 0)),
        out_shape=jax.ShapeDtypeStruct((n, 2 * _DIM), jnp.float32),
        compiler_params=pltpu.CompilerParams(
            dimension_semantics=("arbitrary",)),
    )(x8, s2)


def kernel(x, weights, scales):
    batch, hist = x.shape
    b_flat = batch * hist
    groups_total = b_flat // _GROUP
    idx2d = x.reshape(groups_total, _GROUP)
    rows8, svals = _sc_gather(weights, idx2d, scales)
    x8 = rows8.reshape(b_flat // 2, 2 * _DIM)
    s2 = svals.reshape(b_flat // 2, 2)
    out = _tc_dequant(x8, s2)
    return out.reshape(batch, hist, _DIM)
